# baseline (speedup 1.0000x reference)
import functools

import jax
import jax.numpy as jnp
from jax import lax
from jax.experimental import pallas as pl
from jax.experimental.pallas import tpu as pltpu
from jax.experimental.pallas import tpu_sc as plsc

N_USERS = 10000
N_ARTISTS = 10000
N_EDGES = 320000
D = 128
N_LAYERS = 2

NC = 2
NS = 16
NW = NC * NS
CHUNK = 64
CPT0 = 64
CPT1 = 256
CPT = 160
E_PAD = NS * (CPT0 + CPT1) * CHUNK
R_PAD = 10240
RPT = R_PAD // NS

_MESH = plsc.VectorSubcoreMesh(core_axis_name="c", subcore_axis_name="s",
                               num_cores=NC, num_subcores=NS)


NROW = 4
NDST = 8
GROUP = 8


def _conv_body(table, srcr, dstr, zrows, part, src_r, dst_r, rows_v, acc,
               *sems):
    sem_g = sems[:NROW]
    sem_s = sems[NROW:2 * NROW]
    sem_d = sems[2 * NROW:2 * NROW + NDST]
    sem_e = sems[2 * NROW + NDST:]
    c = lax.axis_index("c")
    s = lax.axis_index("s")
    w = s * NC + c

    pltpu.sync_copy(zrows.at[pl.ds(s * RPT, RPT)], acc.at[pl.ds(s * RPT, RPT)])
    plsc.subcore_barrier()

    def run(cpt, base):
        def start_gather(be, br):
            pltpu.async_copy(table.at[src_r.at[be]], rows_v.at[br], sem_g[br])

        def wait_gather(be, br):
            pltpu.make_async_copy(table.at[src_r.at[be]], rows_v.at[br],
                                  sem_g[br]).wait()

        def start_src(ci, be):
            pltpu.async_copy(srcr.at[base + ci], src_r.at[be], sem_e[be])

        def wait_src(ci, be):
            pltpu.make_async_copy(srcr.at[base + ci], src_r.at[be],
                                  sem_e[be]).wait()

        def start_scat(br, bd):
            pltpu.async_copy(rows_v.at[br], acc.at[dst_r.at[bd]], sem_s[br],
                             add=True)

        def wait_scat(br, bd):
            pltpu.make_async_copy(rows_v.at[br], acc.at[dst_r.at[bd]],
                                  sem_s[br]).wait()

        def start_dst(ci, bd):
            pltpu.async_copy(dstr.at[base + ci], dst_r.at[bd], sem_d[bd])

        def wait_dst(ci, bd):
            pltpu.make_async_copy(dstr.at[base + ci], dst_r.at[bd],
                                  sem_d[bd]).wait()

        def step(ci, b, first=False, more_gather=True, more_dst=True):
            br = b % NROW
            bd = b % NDST
            if not first:
                wait_scat((b - 1) % NROW, (b - 1) % NDST)
            if more_gather:
                wait_src(ci + NROW - 1, (b + NROW - 1) % NDST)
                start_gather((b + NROW - 1) % NDST, (b + NROW - 1) % NROW)
            wait_gather(b % NDST, br)
            wait_dst(ci, bd)
            start_scat(br, bd)
            if more_dst:
                start_dst(ci + NDST - 1, (b + NDST - 1) % NDST)
                start_src(ci + NDST - 1, (b + NDST - 1) % NDST)

        for ci in range(NDST - 1):
            start_dst(ci, ci)
            start_src(ci, ci)
        for ci in range(NROW - 1):
            wait_src(ci, ci)
            start_gather(ci, ci)
        for b in range(GROUP):
            step(b, b, first=(b == 0))

        def group(g, carry):
            ci0 = g * GROUP
            for b in range(GROUP):
                step(ci0 + b, b)
            return carry

        lax.fori_loop(1, cpt // GROUP - 1, group, 0)
        for b in range(GROUP):
            ci = cpt - GROUP + b
            step(ci, b, more_gather=ci + NROW - 1 < cpt,
                 more_dst=ci + NDST - 1 < cpt)
        wait_scat((cpt - 1) % NROW, (cpt - 1) % NDST)

    @pl.when(c == 0)
    def _():
        run(CPT0, s * CPT0)

    @pl.when(c == 1)
    def _():
        run(CPT1, NS * CPT0 + s * CPT1)

    plsc.subcore_barrier()

    pltpu.sync_copy(acc.at[pl.ds(s * RPT, RPT)],
                    part.at[c].at[pl.ds(s * RPT, RPT)])


_conv = pl.kernel(
    _conv_body,
    out_type=[jax.ShapeDtypeStruct((NC, R_PAD, D), jnp.float32)],
    mesh=_MESH,
    scratch_types=[
        pltpu.VMEM((NDST, CHUNK), jnp.int32),
        pltpu.VMEM((NDST, CHUNK), jnp.int32),
        pltpu.VMEM((NROW, CHUNK, D), jnp.float32),
        pltpu.VMEM_SHARED((R_PAD, D), jnp.float32),
    ] + [pltpu.SemaphoreType.DMA] * (2 * NROW + 2 * NDST),
    name="lgcn_conv",
)


NSEM = 4


def _cnt_body(dstr, zrows, ones_h, cnt, dst_v, ones_v, acc, *sems):
    c = lax.axis_index("c")
    s = lax.axis_index("s")
    w = s * NC + c

    pltpu.sync_copy(zrows.at[pl.ds(s * RPT, RPT)], acc.at[pl.ds(s * RPT, RPT)])
    pltpu.sync_copy(ones_h, ones_v)
    pltpu.sync_copy(dstr.at[pl.ds(w * CPT, CPT)], dst_v)
    plsc.subcore_barrier()

    def start_scat(ci, k):
        pltpu.async_copy(ones_v, acc.at[dst_v.at[ci]], sems[k], add=True)

    def wait_scat(ci, k):
        pltpu.make_async_copy(ones_v, acc.at[dst_v.at[ci]], sems[k]).wait()

    for b in range(NSEM):
        if b == NSEM - 1:
            wait_scat(0, 0)
        start_scat(b, b)

    def group(g, carry):
        ci0 = g * NSEM
        for b in range(NSEM):
            ci = ci0 + b
            wait_scat(ci - NSEM + 1, (b + 1) % NSEM)
            start_scat(ci, b)
        return carry

    lax.fori_loop(1, CPT // NSEM, group, 0)
    for b in range(1, NSEM):
        wait_scat(CPT - NSEM + b, b)
    plsc.subcore_barrier()

    pltpu.sync_copy(acc.at[pl.ds(s * RPT, RPT)],
                    cnt.at[c].at[pl.ds(s * RPT, RPT)])


_cnt = pl.kernel(
    _cnt_body,
    out_type=[jax.ShapeDtypeStruct((NC, R_PAD, D), jnp.float32)],
    mesh=_MESH,
    scratch_types=[
        pltpu.VMEM((CPT, CHUNK), jnp.int32),
        pltpu.VMEM((CHUNK, D), jnp.float32),
        pltpu.VMEM_SHARED((R_PAD, D), jnp.float32),
    ] + [pltpu.SemaphoreType.DMA] * NSEM,
    name="lgcn_cnt",
)


_BR = 1024


def _combine_body(part, cnt, out):
    p = part[0] + part[1]
    cm = cnt[0, :, 0:1] + cnt[1, :, 0:1]
    out[...] = p / jnp.maximum(cm, 1.0)


def _combine(part, cnt):
    return pl.pallas_call(
        _combine_body,
        grid=(R_PAD // _BR,),
        in_specs=[
            pl.BlockSpec((NC, _BR, D), lambda i: (0, i, 0)),
            pl.BlockSpec((NC, _BR, D), lambda i: (0, i, 0)),
        ],
        out_specs=pl.BlockSpec((_BR, D), lambda i: (i, 0)),
        out_shape=jax.ShapeDtypeStruct((R_PAD, D), jnp.float32),
    )(part, cnt)


def _finalize_body(emit_x2, part, cnt, x0, x1, *outs):
    p = part[0] + part[1]
    cm = cnt[0, :, 0:1] + cnt[1, :, 0:1]
    x2 = p / jnp.maximum(cm, 1.0)
    outs[0][...] = (x0[...] + x1[...] + x2) * (1.0 / (N_LAYERS + 1))
    if emit_x2:
        outs[1][...] = x2


def _finalize(part, cnt, x0, x1, emit_x2):
    n_out = 2 if emit_x2 else 1
    out_shape = [jax.ShapeDtypeStruct((R_PAD, D), jnp.float32)] * n_out
    return pl.pallas_call(
        functools.partial(_finalize_body, emit_x2),
        grid=(R_PAD // _BR,),
        in_specs=[
            pl.BlockSpec((NC, _BR, D), lambda i: (0, i, 0)),
            pl.BlockSpec((NC, _BR, D), lambda i: (0, i, 0)),
            pl.BlockSpec((_BR, D), lambda i: (i, 0)),
            pl.BlockSpec((_BR, D), lambda i: (i, 0)),
        ],
        out_specs=[pl.BlockSpec((_BR, D), lambda i: (i, 0))] * n_out,
        out_shape=out_shape,
    )(part, cnt, x0, x1)


def _prep_edges(ei):
    src = ei[0].astype(jnp.int32)
    dst = ei[1].astype(jnp.int32)
    pad = E_PAD - N_EDGES
    src = jnp.pad(src, (0, pad), constant_values=0)
    dst = jnp.pad(dst, (0, pad), constant_values=R_PAD - 1)
    return src.reshape(NW * CPT, CHUNK), dst.reshape(NW * CPT, CHUNK)


def kernel(x_users, x_artists, edge_index_a2u, edge_index_u2a):
    xu0 = jnp.pad(x_users.astype(jnp.float32), ((0, R_PAD - N_USERS), (0, 0)))
    xa0 = jnp.pad(x_artists.astype(jnp.float32),
                  ((0, R_PAD - N_ARTISTS), (0, 0)))
    src_au, dst_au = _prep_edges(edge_index_a2u)
    src_ua, dst_ua = _prep_edges(edge_index_u2a)
    zrows = jnp.zeros((R_PAD, D), jnp.float32)
    ones_blk = jnp.ones((CHUNK, D), jnp.float32)

    (cnt_u,) = _cnt(dst_au, zrows, ones_blk)
    (cnt_a,) = _cnt(dst_ua, zrows, ones_blk)
    (part_u,) = _conv(xa0, src_au, dst_au, zrows)
    xu1 = _combine(part_u, cnt_u)
    (part_a,) = _conv(xu1, src_ua, dst_ua, zrows)
    xa1 = _combine(part_a, cnt_a)
    (part_u2,) = _conv(xa1, src_au, dst_au, zrows)
    final_u, xu2 = _finalize(part_u2, cnt_u, xu0, xu1, True)
    (part_a2,) = _conv(xu2, src_ua, dst_ua, zrows)
    (final_a,) = _finalize(part_a2, cnt_a, xa0, xa1, False)

    return (final_u[:N_USERS], final_a[:N_ARTISTS])


# speedup vs baseline: 1.0823x; 1.0823x over previous
import functools

import jax
import jax.numpy as jnp
from jax import lax
from jax.experimental import pallas as pl
from jax.experimental.pallas import tpu as pltpu
from jax.experimental.pallas import tpu_sc as plsc

N_USERS = 10000
N_ARTISTS = 10000
N_EDGES = 320000
D = 128
N_LAYERS = 2

NC = 2
NS = 16
NW = NC * NS
CHUNK = 64
CPT0 = 256
CPT1 = 64
CPT = 160
E_PAD = NS * (CPT0 + CPT1) * CHUNK
R_PAD = 10240
RPT = R_PAD // NS

_MESH = plsc.VectorSubcoreMesh(core_axis_name="c", subcore_axis_name="s",
                               num_cores=NC, num_subcores=NS)


NROW = 4
NDST = 8
GROUP = 8


def _conv_body(table, srcr, dstr, zrows, part, src_r, dst_r, rows_v, acc,
               *sems):
    sem_g = sems[:NROW]
    sem_s = sems[NROW:2 * NROW]
    sem_d = sems[2 * NROW:2 * NROW + NDST]
    sem_e = sems[2 * NROW + NDST:]
    c = lax.axis_index("c")
    s = lax.axis_index("s")
    w = s * NC + c

    pltpu.sync_copy(zrows.at[pl.ds(s * RPT, RPT)], acc.at[pl.ds(s * RPT, RPT)])
    plsc.subcore_barrier()

    def run(cpt, base):
        def start_gather(be, br):
            pltpu.async_copy(table.at[src_r.at[be]], rows_v.at[br], sem_g[br])

        def wait_gather(be, br):
            pltpu.make_async_copy(table.at[src_r.at[be]], rows_v.at[br],
                                  sem_g[br]).wait()

        def start_src(ci, be):
            pltpu.async_copy(srcr.at[base + ci], src_r.at[be], sem_e[be])

        def wait_src(ci, be):
            pltpu.make_async_copy(srcr.at[base + ci], src_r.at[be],
                                  sem_e[be]).wait()

        def start_scat(br, bd):
            pltpu.async_copy(rows_v.at[br], acc.at[dst_r.at[bd]], sem_s[br],
                             add=True)

        def wait_scat(br, bd):
            pltpu.make_async_copy(rows_v.at[br], acc.at[dst_r.at[bd]],
                                  sem_s[br]).wait()

        def start_dst(ci, bd):
            pltpu.async_copy(dstr.at[base + ci], dst_r.at[bd], sem_d[bd])

        def wait_dst(ci, bd):
            pltpu.make_async_copy(dstr.at[base + ci], dst_r.at[bd],
                                  sem_d[bd]).wait()

        def step(ci, b, first=False, more_gather=True, more_dst=True):
            br = b % NROW
            bd = b % NDST
            if not first:
                wait_scat((b - 1) % NROW, (b - 1) % NDST)
            if more_gather:
                wait_src(ci + NROW - 1, (b + NROW - 1) % NDST)
                start_gather((b + NROW - 1) % NDST, (b + NROW - 1) % NROW)
            wait_gather(b % NDST, br)
            wait_dst(ci, bd)
            start_scat(br, bd)
            if more_dst:
                start_dst(ci + NDST - 1, (b + NDST - 1) % NDST)
                start_src(ci + NDST - 1, (b + NDST - 1) % NDST)

        for ci in range(NDST - 1):
            start_dst(ci, ci)
            start_src(ci, ci)
        for ci in range(NROW - 1):
            wait_src(ci, ci)
            start_gather(ci, ci)
        for b in range(GROUP):
            step(b, b, first=(b == 0))

        def group(g, carry):
            ci0 = g * GROUP
            for b in range(GROUP):
                step(ci0 + b, b)
            return carry

        lax.fori_loop(1, cpt // GROUP - 1, group, 0)
        for b in range(GROUP):
            ci = cpt - GROUP + b
            step(ci, b, more_gather=ci + NROW - 1 < cpt,
                 more_dst=ci + NDST - 1 < cpt)
        wait_scat((cpt - 1) % NROW, (cpt - 1) % NDST)

    @pl.when(c == 0)
    def _():
        run(CPT0, s * CPT0)

    @pl.when(c == 1)
    def _():
        run(CPT1, NS * CPT0 + s * CPT1)

    plsc.subcore_barrier()

    pltpu.sync_copy(acc.at[pl.ds(s * RPT, RPT)],
                    part.at[c].at[pl.ds(s * RPT, RPT)])


_conv = pl.kernel(
    _conv_body,
    out_type=[jax.ShapeDtypeStruct((NC, R_PAD, D), jnp.float32)],
    mesh=_MESH,
    scratch_types=[
        pltpu.VMEM((NDST, CHUNK), jnp.int32),
        pltpu.VMEM((NDST, CHUNK), jnp.int32),
        pltpu.VMEM((NROW, CHUNK, D), jnp.float32),
        pltpu.VMEM_SHARED((R_PAD, D), jnp.float32),
    ] + [pltpu.SemaphoreType.DMA] * (2 * NROW + 2 * NDST),
    name="lgcn_conv",
)


NSEM = 4


def _cnt_body(dstr, zrows, ones_h, cnt, dst_v, ones_v, acc, *sems):
    c = lax.axis_index("c")
    s = lax.axis_index("s")
    w = s * NC + c

    pltpu.sync_copy(zrows.at[pl.ds(s * RPT, RPT)], acc.at[pl.ds(s * RPT, RPT)])
    pltpu.sync_copy(ones_h, ones_v)
    pltpu.sync_copy(dstr.at[pl.ds(w * CPT, CPT)], dst_v)
    plsc.subcore_barrier()

    def start_scat(ci, k):
        pltpu.async_copy(ones_v, acc.at[dst_v.at[ci]], sems[k], add=True)

    def wait_scat(ci, k):
        pltpu.make_async_copy(ones_v, acc.at[dst_v.at[ci]], sems[k]).wait()

    for b in range(NSEM):
        if b == NSEM - 1:
            wait_scat(0, 0)
        start_scat(b, b)

    def group(g, carry):
        ci0 = g * NSEM
        for b in range(NSEM):
            ci = ci0 + b
            wait_scat(ci - NSEM + 1, (b + 1) % NSEM)
            start_scat(ci, b)
        return carry

    lax.fori_loop(1, CPT // NSEM, group, 0)
    for b in range(1, NSEM):
        wait_scat(CPT - NSEM + b, b)
    plsc.subcore_barrier()

    pltpu.sync_copy(acc.at[pl.ds(s * RPT, RPT)],
                    cnt.at[c].at[pl.ds(s * RPT, RPT)])


_cnt = pl.kernel(
    _cnt_body,
    out_type=[jax.ShapeDtypeStruct((NC, R_PAD, D), jnp.float32)],
    mesh=_MESH,
    scratch_types=[
        pltpu.VMEM((CPT, CHUNK), jnp.int32),
        pltpu.VMEM((CHUNK, D), jnp.float32),
        pltpu.VMEM_SHARED((R_PAD, D), jnp.float32),
    ] + [pltpu.SemaphoreType.DMA] * NSEM,
    name="lgcn_cnt",
)


_BR = 1024


def _combine_body(part, cnt, out):
    p = part[0] + part[1]
    cm = cnt[0, :, 0:1] + cnt[1, :, 0:1]
    out[...] = p / jnp.maximum(cm, 1.0)


def _combine(part, cnt):
    return pl.pallas_call(
        _combine_body,
        grid=(R_PAD // _BR,),
        in_specs=[
            pl.BlockSpec((NC, _BR, D), lambda i: (0, i, 0)),
            pl.BlockSpec((NC, _BR, D), lambda i: (0, i, 0)),
        ],
        out_specs=pl.BlockSpec((_BR, D), lambda i: (i, 0)),
        out_shape=jax.ShapeDtypeStruct((R_PAD, D), jnp.float32),
    )(part, cnt)


def _finalize_body(emit_x2, part, cnt, x0, x1, *outs):
    p = part[0] + part[1]
    cm = cnt[0, :, 0:1] + cnt[1, :, 0:1]
    x2 = p / jnp.maximum(cm, 1.0)
    outs[0][...] = (x0[...] + x1[...] + x2) * (1.0 / (N_LAYERS + 1))
    if emit_x2:
        outs[1][...] = x2


def _finalize(part, cnt, x0, x1, emit_x2):
    n_out = 2 if emit_x2 else 1
    out_shape = [jax.ShapeDtypeStruct((R_PAD, D), jnp.float32)] * n_out
    return pl.pallas_call(
        functools.partial(_finalize_body, emit_x2),
        grid=(R_PAD // _BR,),
        in_specs=[
            pl.BlockSpec((NC, _BR, D), lambda i: (0, i, 0)),
            pl.BlockSpec((NC, _BR, D), lambda i: (0, i, 0)),
            pl.BlockSpec((_BR, D), lambda i: (i, 0)),
            pl.BlockSpec((_BR, D), lambda i: (i, 0)),
        ],
        out_specs=[pl.BlockSpec((_BR, D), lambda i: (i, 0))] * n_out,
        out_shape=out_shape,
    )(part, cnt, x0, x1)


def _prep_edges(ei):
    src = ei[0].astype(jnp.int32)
    dst = ei[1].astype(jnp.int32)
    pad = E_PAD - N_EDGES
    src = jnp.pad(src, (0, pad), constant_values=0)
    dst = jnp.pad(dst, (0, pad), constant_values=R_PAD - 1)
    return src.reshape(NW * CPT, CHUNK), dst.reshape(NW * CPT, CHUNK)


def kernel(x_users, x_artists, edge_index_a2u, edge_index_u2a):
    xu0 = jnp.pad(x_users.astype(jnp.float32), ((0, R_PAD - N_USERS), (0, 0)))
    xa0 = jnp.pad(x_artists.astype(jnp.float32),
                  ((0, R_PAD - N_ARTISTS), (0, 0)))
    src_au, dst_au = _prep_edges(edge_index_a2u)
    src_ua, dst_ua = _prep_edges(edge_index_u2a)
    zrows = jnp.zeros((R_PAD, D), jnp.float32)
    ones_blk = jnp.ones((CHUNK, D), jnp.float32)

    (cnt_u,) = _cnt(dst_au, zrows, ones_blk)
    (cnt_a,) = _cnt(dst_ua, zrows, ones_blk)
    (part_u,) = _conv(xa0, src_au, dst_au, zrows)
    xu1 = _combine(part_u, cnt_u)
    (part_a,) = _conv(xu1, src_ua, dst_ua, zrows)
    xa1 = _combine(part_a, cnt_a)
    (part_u2,) = _conv(xa1, src_au, dst_au, zrows)
    final_u, xu2 = _finalize(part_u2, cnt_u, xu0, xu1, True)
    (part_a2,) = _conv(xu2, src_ua, dst_ua, zrows)
    (final_a,) = _finalize(part_a2, cnt_a, xa0, xa1, False)

    return (final_u[:N_USERS], final_a[:N_ARTISTS])
